# baseline (speedup 1.0000x reference)
import functools

import jax
import jax.numpy as jnp
from jax import lax
from jax.experimental import pallas as pl
from jax.experimental.pallas import tpu as pltpu
from jax.experimental.pallas import tpu_sc as plsc

N_NODES = 10000
N_EDGES = 320000
D = 128
DH = D // 2

NC = 2
NS = 16
EPT = N_EDGES // NS
K = 80
NCHUNK = EPT // K
RING = 4
ROWS_PT = 624
ROWS_TAIL = N_NODES - NS * ROWS_PT


def _sc_aggregate(embs_halves, rows, cols, vals, zeros):
    mesh = plsc.VectorSubcoreMesh(core_axis_name="c", subcore_axis_name="s")

    @functools.partial(
        pl.kernel,
        out_type=jax.ShapeDtypeStruct((NC, N_NODES, DH), jnp.float32),
        mesh=mesh,
        compiler_params=pltpu.CompilerParams(use_tc_tiling_on_sc=False),
        scratch_types=[
            pltpu.VMEM_SHARED((N_NODES, DH), jnp.float32),
            pltpu.VMEM_SHARED((N_NODES, DH), jnp.float32),
            pltpu.VMEM((RING, K), jnp.int32),
            pltpu.VMEM((RING, K), jnp.int32),
            pltpu.VMEM((RING, K), jnp.float32),
            pltpu.VMEM((RING, K, DH), jnp.float32),
            pltpu.SemaphoreType.DMA((RING,)),
            pltpu.SemaphoreType.DMA((RING,)),
            pltpu.SemaphoreType.DMA((RING,)),
        ],
    )
    def agg_kernel(embs_hbm, rows_hbm, cols_hbm, vals_hbm, zeros_hbm,
                   partial_hbm, embs_sp, agg_sh, colb, rowb, valb, gbuf,
                   isem, gsem, ssem):
        cid = lax.axis_index("c")
        sid = lax.axis_index("s")
        base = sid * EPT

        r0 = pl.multiple_of(sid * ROWS_PT, 8)
        pltpu.sync_copy(zeros_hbm.at[pl.ds(r0, ROWS_PT)],
                        agg_sh.at[pl.ds(r0, ROWS_PT)])
        pltpu.sync_copy(embs_hbm.at[cid, pl.ds(r0, ROWS_PT)],
                        embs_sp.at[pl.ds(r0, ROWS_PT)])

        @pl.when(sid == 0)
        def _():
            pltpu.sync_copy(zeros_hbm.at[pl.ds(NS * ROWS_PT, ROWS_TAIL)],
                            agg_sh.at[pl.ds(NS * ROWS_PT, ROWS_TAIL)])
            pltpu.sync_copy(embs_hbm.at[cid, pl.ds(NS * ROWS_PT, ROWS_TAIL)],
                            embs_sp.at[pl.ds(NS * ROWS_PT, ROWS_TAIL)])

        plsc.subcore_barrier()

        def idx_start(j, b):
            off = pl.multiple_of(base + j * K, 8)
            pltpu.async_copy(cols_hbm.at[pl.ds(off, K)], colb.at[b], isem.at[b])
            pltpu.async_copy(rows_hbm.at[pl.ds(off, K)], rowb.at[b], isem.at[b])
            pltpu.async_copy(vals_hbm.at[pl.ds(off, K)], valb.at[b], isem.at[b])

        def idx_wait(j, b):
            off = pl.multiple_of(base + j * K, 8)
            pltpu.make_async_copy(cols_hbm.at[pl.ds(off, K)], colb.at[b],
                                  isem.at[b]).wait()
            pltpu.make_async_copy(rows_hbm.at[pl.ds(off, K)], rowb.at[b],
                                  isem.at[b]).wait()
            pltpu.make_async_copy(vals_hbm.at[pl.ds(off, K)], valb.at[b],
                                  isem.at[b]).wait()

        def gather_start(b):
            pltpu.async_copy(embs_sp.at[colb.at[b]], gbuf.at[b], gsem.at[b])

        def gather_wait(b):
            pltpu.make_async_copy(embs_sp.at[colb.at[b]], gbuf.at[b],
                                  gsem.at[b]).wait()

        def scat_start(b):
            pltpu.async_copy(gbuf.at[b], agg_sh.at[rowb.at[b]],
                             ssem.at[b], add=True)

        def scat_wait(b):
            pltpu.make_async_copy(gbuf.at[b], agg_sh.at[rowb.at[b]],
                                  ssem.at[b]).wait()

        def scale(b):
            @plsc.parallel_loop(0, K // 16, 1, unroll=1)
            def group_body(g):
                o = pl.multiple_of(g * 16, 8)
                vvec = valb[b, pl.ds(o, 16)]
                nd = DH // 16
                for l in range(16):
                    v = vvec[l]
                    e = g * 16 + l
                    xs = [gbuf[b, e, pl.ds(d * 16, 16)] for d in range(nd)]
                    for d in range(nd):
                        gbuf[b, e, pl.ds(d * 16, 16)] = xs[d] * v

        idx_start(0, 0)
        idx_start(1, 1)
        idx_start(2, 2)
        idx_wait(0, 0)
        gather_start(0)
        idx_wait(1, 1)
        gather_start(1)

        def step(j, b):
            @pl.when(j >= 1)
            def _():
                scat_wait((j - 1) % RING)

            @pl.when(j + 3 < NCHUNK)
            def _():
                idx_start(j + 3, (b + 3) % RING)

            @pl.when(j + 2 < NCHUNK)
            def _():
                idx_wait(j + 2, (b + 2) % RING)
                gather_start((b + 2) % RING)

            gather_wait(b)
            scale(b)
            scat_start(b)

        def quad_body(q, carry):
            for b in range(RING):
                j = q * RING + b
                step(j, b)
            return carry

        lax.fori_loop(0, NCHUNK // RING, quad_body, 0, unroll=False)

        step(NCHUNK - 2, (NCHUNK - 2) % RING)
        step(NCHUNK - 1, (NCHUNK - 1) % RING)
        scat_wait((NCHUNK - 1) % RING)
        plsc.subcore_barrier()

        pltpu.sync_copy(agg_sh.at[pl.ds(r0, ROWS_PT)],
                        partial_hbm.at[cid, pl.ds(r0, ROWS_PT)])

        @pl.when(sid == 0)
        def _():
            pltpu.sync_copy(agg_sh.at[pl.ds(NS * ROWS_PT, ROWS_TAIL)],
                            partial_hbm.at[cid, pl.ds(NS * ROWS_PT, ROWS_TAIL)])

    return agg_kernel(embs_halves, rows, cols, vals, zeros)


def _mm_body(p_ref, w_ref, o_ref):
    lo = jnp.dot(p_ref[0], w_ref[0], preferred_element_type=jnp.float32)
    hi = jnp.dot(p_ref[1], w_ref[1], preferred_element_type=jnp.float32)
    o_ref[...] = jnp.maximum(lo + hi, 0.0)


def _tc_project(partial, W):
    R = 1000
    w2 = W.reshape(NC, DH, D)
    return pl.pallas_call(
        _mm_body,
        grid=(N_NODES // R,),
        in_specs=[
            pl.BlockSpec((NC, R, DH), lambda i: (0, i, 0)),
            pl.BlockSpec((NC, DH, D), lambda i: (0, 0, 0)),
        ],
        out_specs=pl.BlockSpec((R, D), lambda i: (i, 0)),
        out_shape=jax.ShapeDtypeStruct((N_NODES, D), jnp.float32),
    )(partial, w2)


def kernel(node_embs, edge_vals, W, edge_index):
    rows = edge_index[0]
    cols = edge_index[1]
    embs_halves = node_embs.reshape(N_NODES, NC, DH).transpose(1, 0, 2)
    zeros = jnp.zeros((N_NODES, DH), jnp.float32)
    partial = _sc_aggregate(embs_halves, rows, cols, edge_vals, zeros)
    return _tc_project(partial, W)


# speedup vs baseline: 1.0401x; 1.0401x over previous
import functools

import jax
import jax.numpy as jnp
from jax import lax
from jax.experimental import pallas as pl
from jax.experimental.pallas import tpu as pltpu
from jax.experimental.pallas import tpu_sc as plsc

N_NODES = 10000
N_EDGES = 320000
D = 128

NC = 2
NS = 16
NW = NC * NS
EPT = N_EDGES // NW
K = 80
NCHUNK = EPT // K
RING = 4
ROWS_PT = 624
ROWS_TAIL = N_NODES - NS * ROWS_PT


def _sc_aggregate(node_embs, edge_index, vals):
    mesh = plsc.VectorSubcoreMesh(core_axis_name="c", subcore_axis_name="s")

    @functools.partial(
        pl.kernel,
        out_type=jax.ShapeDtypeStruct((NC, N_NODES, D), jnp.float32),
        mesh=mesh,
        compiler_params=pltpu.CompilerParams(use_tc_tiling_on_sc=False,
                                             needs_layout_passes=False),
        scratch_types=[
            pltpu.VMEM_SHARED((N_NODES, D), jnp.float32),
            pltpu.VMEM((RING, K), jnp.int32),
            pltpu.VMEM((RING, K), jnp.int32),
            pltpu.VMEM((RING, K), jnp.float32),
            pltpu.VMEM((RING, K, D // 2), jnp.int32),
            pltpu.VMEM((2, K, D), jnp.float32),
            pltpu.SemaphoreType.DMA((RING,)),
            pltpu.SemaphoreType.DMA((RING,)),
            pltpu.SemaphoreType.DMA((2,)),
        ],
    )
    def agg_kernel(embs_hbm, ei_hbm, vals_hbm,
                   partial_hbm, agg_sh, colb, rowb, valb, gbuf, sbuf,
                   isem, gsem, ssem):
        cid = lax.axis_index("c")
        sid = lax.axis_index("s")
        wid = cid * NS + sid
        base = wid * EPT

        def zfill(i, carry):
            for d in range(D // 16):
                sbuf[0, i, pl.ds(d * 16, 16)] = jnp.zeros((16,), jnp.float32)
            return carry

        lax.fori_loop(0, K, zfill, 0, unroll=False)
        r0 = pl.multiple_of(sid * ROWS_PT, 8)
        for z in range(ROWS_PT // K):
            pltpu.sync_copy(sbuf.at[0],
                            agg_sh.at[pl.ds(r0 + z * K, K)])
        pltpu.sync_copy(sbuf.at[0, pl.ds(0, ROWS_PT - (ROWS_PT // K) * K)],
                        agg_sh.at[pl.ds(r0 + (ROWS_PT // K) * K,
                                        ROWS_PT - (ROWS_PT // K) * K)])

        @pl.when(sid == 0)
        def _():
            pltpu.sync_copy(sbuf.at[0, pl.ds(0, ROWS_TAIL)],
                            agg_sh.at[pl.ds(NS * ROWS_PT, ROWS_TAIL)])

        plsc.subcore_barrier()

        def idx_start(j, b):
            off = pl.multiple_of(base + j * K, 8)
            pltpu.async_copy(ei_hbm.at[pl.ds(off, K)], rowb.at[b],
                             isem.at[b])
            pltpu.async_copy(ei_hbm.at[pl.ds(N_EDGES + off, K)], colb.at[b],
                             isem.at[b])
            pltpu.async_copy(vals_hbm.at[pl.ds(off, K)], valb.at[b], isem.at[b])

        def idx_wait(j, b):
            off = pl.multiple_of(base + j * K, 8)
            pltpu.make_async_copy(ei_hbm.at[pl.ds(off, K)], rowb.at[b],
                                  isem.at[b]).wait()
            pltpu.make_async_copy(ei_hbm.at[pl.ds(N_EDGES + off, K)], colb.at[b],
                                  isem.at[b]).wait()
            pltpu.make_async_copy(vals_hbm.at[pl.ds(off, K)], valb.at[b],
                                  isem.at[b]).wait()

        def gather_start(b):
            pltpu.async_copy(embs_hbm.at[colb.at[b]], gbuf.at[b], gsem.at[b])

        def gather_wait(b):
            pltpu.make_async_copy(embs_hbm.at[colb.at[b]], gbuf.at[b],
                                  gsem.at[b]).wait()

        def scat_start(b, s):
            pltpu.async_copy(sbuf.at[s], agg_sh.at[rowb.at[b]],
                             ssem.at[s], add=True)

        def scat_wait(b, s):
            pltpu.make_async_copy(sbuf.at[s], agg_sh.at[rowb.at[b]],
                                  ssem.at[s]).wait()

        def scale(b, s):
            @plsc.parallel_loop(0, K // 16, 1, unroll=1)
            def group_body(g):
                o = pl.multiple_of(g * 16, 8)
                vvec = valb[b, pl.ds(o, 16)]
                for l in range(16):
                    v = vvec[l]
                    e = g * 16 + l
                    xs = []
                    for h in range(D // 32):
                        q = gbuf[b, e, pl.ds(h * 16, 16)]
                        ab = plsc.bitcast(q, jnp.bfloat16)
                        lo, hi = plsc.unpack(
                            ab, format=plsc.PackFormat.INTERLEAVED,
                            preferred_element_type=jnp.float32)
                        xs.extend([lo, hi])
                    for d in range(D // 16):
                        sbuf[s, e, pl.ds(d * 16, 16)] = xs[d] * v

        idx_start(0, 0)
        idx_start(1, 1)
        idx_wait(0, 0)
        gather_start(0)

        def step(j, b, s):
            @pl.when(j >= 1)
            def _():
                scat_wait((j - 1) % RING, 1 - s)

            @pl.when(j + 2 < NCHUNK)
            def _():
                idx_start(j + 2, (b + 2) % RING)

            @pl.when(j + 1 < NCHUNK)
            def _():
                idx_wait(j + 1, (b + 1) % RING)
                gather_start((b + 1) % RING)

            gather_wait(b)
            scale(b, s)
            scat_start(b, s)

        def quad_body(q, carry):
            for b in range(RING):
                j = q * RING + b
                step(j, b, b % 2)
            return carry

        lax.fori_loop(0, NCHUNK // RING, quad_body, 0, unroll=False)

        step(NCHUNK - 1, (NCHUNK - 1) % RING, (NCHUNK - 1) % 2)

        scat_wait((NCHUNK - 1) % RING, (NCHUNK - 1) % 2)
        plsc.subcore_barrier()

        pltpu.sync_copy(agg_sh.at[pl.ds(r0, ROWS_PT)],
                        partial_hbm.at[cid, pl.ds(r0, ROWS_PT)])

        @pl.when(sid == 0)
        def _():
            pltpu.sync_copy(agg_sh.at[pl.ds(NS * ROWS_PT, ROWS_TAIL)],
                            partial_hbm.at[cid, pl.ds(NS * ROWS_PT, ROWS_TAIL)])

    return agg_kernel(node_embs, edge_index.reshape(-1), vals)


def _mm_body(p_ref, w_ref, o_ref):
    acc = p_ref[0] + p_ref[1]
    o_ref[...] = jnp.maximum(
        jnp.dot(acc, w_ref[...], preferred_element_type=jnp.float32), 0.0)


def _tc_project(partial, W):
    R = 1000
    return pl.pallas_call(
        _mm_body,
        grid=(N_NODES // R,),
        in_specs=[
            pl.BlockSpec((NC, R, D), lambda i: (0, i, 0)),
            pl.BlockSpec((D, D), lambda i: (0, 0)),
        ],
        out_specs=pl.BlockSpec((R, D), lambda i: (i, 0)),
        out_shape=jax.ShapeDtypeStruct((N_NODES, D), jnp.float32),
    )(partial, W)


def kernel(node_embs, edge_vals, W, edge_index):
    embs_packed = lax.bitcast_convert_type(
        node_embs.astype(jnp.bfloat16)
        .reshape(N_NODES, 4, 2, 16)
        .swapaxes(2, 3)
        .reshape(N_NODES, D // 2, 2),
        jnp.int32)
    partial = _sc_aggregate(embs_packed, edge_index, edge_vals)
    return _tc_project(partial, W)


# speedup vs baseline: 1.2733x; 1.2242x over previous
import functools

import jax
import jax.numpy as jnp
from jax import lax
from jax.experimental import pallas as pl
from jax.experimental.pallas import tpu as pltpu
from jax.experimental.pallas import tpu_sc as plsc

N_NODES = 10000
N_EDGES = 320000
D = 128

NC = 2
NS = 16
NW = NC * NS
EPT = N_EDGES // NW
K = 80
NCHUNK = EPT // K
RING = 4
ROWS_PT = 624
ROWS_TAIL = N_NODES - NS * ROWS_PT


def _sc_aggregate(node_embs, edge_index, vals):
    mesh = plsc.VectorSubcoreMesh(core_axis_name="c", subcore_axis_name="s")

    @functools.partial(
        pl.kernel,
        out_type=jax.ShapeDtypeStruct((NC, N_NODES, D), jnp.float32),
        mesh=mesh,
        scratch_types=[
            pltpu.VMEM_SHARED((N_NODES, D), jnp.float32),
            pltpu.VMEM((RING, K), jnp.int32),
            pltpu.VMEM((RING, K), jnp.int32),
            pltpu.VMEM((RING, K), jnp.float32),
            pltpu.VMEM((RING, K, D), jnp.float32),
            pltpu.SemaphoreType.DMA((RING,)),
            pltpu.SemaphoreType.DMA((RING,)),
            pltpu.SemaphoreType.DMA((RING,)),
        ],
    )
    def agg_kernel(embs_hbm, ei_hbm, vals_hbm,
                   partial_hbm, agg_sh, colb, rowb, valb, gbuf,
                   isem, gsem, ssem):
        cid = lax.axis_index("c")
        sid = lax.axis_index("s")
        wid = cid * NS + sid
        base = wid * EPT

        def zfill(i, carry):
            for d in range(D // 16):
                gbuf[0, i, pl.ds(d * 16, 16)] = jnp.zeros((16,), jnp.float32)
            return carry

        lax.fori_loop(0, K, zfill, 0, unroll=False)
        r0 = pl.multiple_of(sid * ROWS_PT, 8)
        for z in range(ROWS_PT // K):
            pltpu.sync_copy(gbuf.at[0],
                            agg_sh.at[pl.ds(r0 + z * K, K)])
        pltpu.sync_copy(gbuf.at[0, pl.ds(0, ROWS_PT - (ROWS_PT // K) * K)],
                        agg_sh.at[pl.ds(r0 + (ROWS_PT // K) * K,
                                        ROWS_PT - (ROWS_PT // K) * K)])

        @pl.when(sid == 0)
        def _():
            pltpu.sync_copy(gbuf.at[0, pl.ds(0, ROWS_TAIL)],
                            agg_sh.at[pl.ds(NS * ROWS_PT, ROWS_TAIL)])

        plsc.subcore_barrier()

        def idx_start(j, b):
            off = pl.multiple_of(base + j * K, 8)
            pltpu.async_copy(ei_hbm.at[pl.ds(off, K)], rowb.at[b],
                             isem.at[b])
            pltpu.async_copy(ei_hbm.at[pl.ds(N_EDGES + off, K)], colb.at[b],
                             isem.at[b])
            pltpu.async_copy(vals_hbm.at[pl.ds(off, K)], valb.at[b], isem.at[b])

        def idx_wait(j, b):
            off = pl.multiple_of(base + j * K, 8)
            pltpu.make_async_copy(ei_hbm.at[pl.ds(off, K)], rowb.at[b],
                                  isem.at[b]).wait()
            pltpu.make_async_copy(ei_hbm.at[pl.ds(N_EDGES + off, K)], colb.at[b],
                                  isem.at[b]).wait()
            pltpu.make_async_copy(vals_hbm.at[pl.ds(off, K)], valb.at[b],
                                  isem.at[b]).wait()

        def gather_start(b):
            pltpu.async_copy(embs_hbm.at[colb.at[b]], gbuf.at[b], gsem.at[b])

        def gather_wait(b):
            pltpu.make_async_copy(embs_hbm.at[colb.at[b]], gbuf.at[b],
                                  gsem.at[b]).wait()

        def scat_start(b):
            pltpu.async_copy(gbuf.at[b], agg_sh.at[rowb.at[b]],
                             ssem.at[b], add=True)

        def scat_wait(b):
            pltpu.make_async_copy(gbuf.at[b], agg_sh.at[rowb.at[b]],
                                  ssem.at[b]).wait()

        def scale(b):
            @plsc.parallel_loop(0, K // 16, 1, unroll=1)
            def group_body(g):
                o = pl.multiple_of(g * 16, 8)
                vvec = valb[b, pl.ds(o, 16)]
                nd = D // 16
                for l in range(16):
                    v = vvec[l]
                    e = g * 16 + l
                    xs = [gbuf[b, e, pl.ds(d * 16, 16)] for d in range(nd)]
                    for d in range(nd):
                        gbuf[b, e, pl.ds(d * 16, 16)] = xs[d] * v

        idx_start(0, 0)
        idx_start(1, 1)
        idx_wait(0, 0)
        gather_start(0)

        def step(j, b):
            @pl.when(j >= 2)
            def _():
                scat_wait((j - 2) % RING)

            @pl.when(j + 2 < NCHUNK)
            def _():
                idx_start(j + 2, (b + 2) % RING)

            @pl.when(j + 1 < NCHUNK)
            def _():
                idx_wait(j + 1, (b + 1) % RING)
                gather_start((b + 1) % RING)

            gather_wait(b)
            scale(b)
            scat_start(b)

        def quad_body(q, carry):
            for b in range(RING):
                j = q * RING + b
                step(j, b)
            return carry

        lax.fori_loop(0, NCHUNK // RING, quad_body, 0, unroll=False)

        step(NCHUNK - 1, (NCHUNK - 1) % RING)

        scat_wait((NCHUNK - 2) % RING)
        scat_wait((NCHUNK - 1) % RING)
        plsc.subcore_barrier()

        pltpu.sync_copy(agg_sh.at[pl.ds(r0, ROWS_PT)],
                        partial_hbm.at[cid, pl.ds(r0, ROWS_PT)])

        @pl.when(sid == 0)
        def _():
            pltpu.sync_copy(agg_sh.at[pl.ds(NS * ROWS_PT, ROWS_TAIL)],
                            partial_hbm.at[cid, pl.ds(NS * ROWS_PT, ROWS_TAIL)])

    return agg_kernel(node_embs, edge_index.reshape(-1), vals)


def _mm_body(p_ref, w_ref, o_ref):
    acc = p_ref[0] + p_ref[1]
    o_ref[...] = jnp.maximum(
        jnp.dot(acc, w_ref[...], preferred_element_type=jnp.float32), 0.0)


def _tc_project(partial, W):
    R = 1000
    return pl.pallas_call(
        _mm_body,
        grid=(N_NODES // R,),
        in_specs=[
            pl.BlockSpec((NC, R, D), lambda i: (0, i, 0)),
            pl.BlockSpec((D, D), lambda i: (0, 0)),
        ],
        out_specs=pl.BlockSpec((R, D), lambda i: (i, 0)),
        out_shape=jax.ShapeDtypeStruct((N_NODES, D), jnp.float32),
    )(partial, W)


def kernel(node_embs, edge_vals, W, edge_index):
    partial = _sc_aggregate(node_embs, edge_index, edge_vals)
    return _tc_project(partial, W)
